# Initial kernel scaffold; baseline (speedup 1.0000x reference)
import jax
import jax.numpy as jnp
from jax.experimental import pallas as pl


def kernel(pos, x, W0, b0):
    raise NotImplementedError("write your pallas kernel here")


# speedup vs baseline: 4.1884x; 4.1884x over previous
import functools

import jax
import jax.numpy as jnp
from jax import lax
from jax.experimental import pallas as pl
from jax.experimental.pallas import tpu as pltpu

N = 10000
C = 125
NPOINT = 2048
RADIUS = 0.2
NSAMPLE = 64
D_OUT = 64

NPAD = 10240
SUB = 8
LANES = NPAD // SUB


def _fps_body(poslin_ref, px_ref, py_ref, pz_ref, npos_ref, dists_ref):
    flat = (lax.broadcasted_iota(jnp.int32, (SUB, LANES), 0) * LANES
            + lax.broadcasted_iota(jnp.int32, (SUB, LANES), 1))
    dists_ref[...] = jnp.where(flat < N, jnp.float32(jnp.inf),
                               jnp.float32(-jnp.inf))

    def body(i, f):
        row = poslin_ref[pl.ds(f, 1), :]
        npos_ref[pl.ds(i, 1), :] = row
        cx = row[:, 0:1]
        cy = row[:, 1:2]
        cz = row[:, 2:3]
        dx = px_ref[...] - cx
        dy = py_ref[...] - cy
        dz = pz_ref[...] - cz
        d = (dx * dx + dy * dy) + dz * dz
        nd = jnp.minimum(dists_ref[...], d)
        dists_ref[...] = nd
        m = jnp.max(nd)
        cand = jnp.where(nd == m, flat, jnp.int32(2**30))
        return jnp.min(cand)

    lax.fori_loop(0, NPOINT, body, jnp.int32(0))


def _fps(pos):
    poslin = jnp.zeros((NPAD, 3), jnp.float32).at[:N].set(pos)
    planes = poslin.T.reshape(3, SUB, LANES)
    return pl.pallas_call(
        _fps_body,
        out_shape=jax.ShapeDtypeStruct((NPOINT, 3), jnp.float32),
        scratch_shapes=[pltpu.VMEM((SUB, LANES), jnp.float32)],
    )(poslin, planes[0], planes[1], planes[2])


def kernel(pos, x, W0, b0):
    npos = _fps(pos)
    z = jnp.concatenate([x, pos], axis=1) @ W0 + b0
    c = npos @ W0[C:]
    d2 = jnp.sum((npos[:, None, :] - pos[None, :, :]) ** 2, axis=-1)
    score = jnp.where(d2 <= RADIUS * RADIUS, -d2, -jnp.inf)
    vals, nbr = lax.top_k(score, NSAMPLE)
    valid = jnp.isfinite(vals)
    m = jnp.max(jnp.where(valid[..., None], z[nbr], -jnp.inf), axis=1)
    new_x = jax.nn.relu(m - c)
    return npos, new_x
